# Initial kernel scaffold; baseline (speedup 1.0000x reference)
import jax
import jax.numpy as jnp
from jax.experimental import pallas as pl


def kernel(in_feat, edge_index, W_m, b_m, W_d, b_d, W1, al1, ar1, b1, W2, al2, ar2, b2):
    raise NotImplementedError("write your pallas kernel here")


# speedup vs baseline: 64.1973x; 64.1973x over previous
import functools

import jax
import jax.numpy as jnp
from jax import lax
from jax.experimental import pallas as pl
from jax.experimental.pallas import tpu as pltpu
from jax.experimental.pallas import tpu_sc as plsc

N = 878
NP = 896
H = 8
HF = 256
FC = 256
E = 28096
D = H * HF

_ROWS_PER_TILE = NP // 32
_TILE_WORDS = _ROWS_PER_TILE * NP
_LANES = 16


def _adj_body(src_hbm, dst_hbm, a_hbm, src_v, dst_v, acc_v):
    wid = lax.axis_index("s") * 2 + lax.axis_index("c")
    lo = wid * _ROWS_PER_TILE

    pltpu.sync_copy(src_hbm, src_v)
    pltpu.sync_copy(dst_hbm, dst_v)

    def _zero(i, _):
        acc_v[pl.ds(i * _LANES, _LANES)] = jnp.zeros((_LANES,), jnp.float32)
        return 0

    lax.fori_loop(0, _TILE_WORDS // _LANES, _zero, 0)

    ones = jnp.ones((_LANES,), jnp.float32)

    def _scatter(i, _):
        d16 = dst_v[pl.ds(i * _LANES, _LANES)]
        s16 = src_v[pl.ds(i * _LANES, _LANES)]
        rel = d16 - lo
        msk = (rel >= 0) & (rel < _ROWS_PER_TILE)
        relc = jnp.clip(rel, 0, _ROWS_PER_TILE - 1)
        plsc.addupdate_scatter(acc_v, [relc * NP + s16], ones, mask=msk)
        return 0

    lax.fori_loop(0, E // _LANES, _scatter, 0)

    pltpu.sync_copy(acc_v, a_hbm.at[pl.ds(wid * _TILE_WORDS, _TILE_WORDS)])


@functools.cache
def _build_adj_fn():
    return functools.partial(
        pl.kernel,
        out_type=jax.ShapeDtypeStruct((NP * NP,), jnp.float32),
        mesh=plsc.VectorSubcoreMesh(core_axis_name="c", subcore_axis_name="s"),
        compiler_params=pltpu.CompilerParams(needs_layout_passes=False),
        scratch_types=[
            pltpu.VMEM((E,), jnp.int32),
            pltpu.VMEM((E,), jnp.int32),
            pltpu.VMEM((_TILE_WORDS,), jnp.float32),
        ],
    )(_adj_body)


def _fc_body(bf_ref, wm_ref, wd_ref, bm_ref, bd_ref, x_ref):
    b = bf_ref[...]
    xm = jnp.dot(b, wm_ref[...], preferred_element_type=jnp.float32) + bm_ref[...]
    xd = jnp.dot(b, wd_ref[...], preferred_element_type=jnp.float32) + bd_ref[...]
    row = lax.broadcasted_iota(jnp.int32, (NP, 1), 0)
    x = jnp.where(row < 495, xm, xd)
    x_ref[...] = jnp.where(row < N, x, 0.0)


def _fc(bf, wm, wd, bm, bd):
    return pl.pallas_call(
        _fc_body,
        out_shape=jax.ShapeDtypeStruct((NP, FC), jnp.float32),
    )(bf, wm, wd, bm, bd)


def _proj_body(x_ref, w_ref, mal_ref, mar_ref, feat_ref, el_ref, er_ref):
    f = jnp.dot(x_ref[...], w_ref[...], preferred_element_type=jnp.float32)
    feat_ref[...] = f
    el_ref[...] = jnp.dot(f, mal_ref[...], preferred_element_type=jnp.float32)
    er_ref[...] = jnp.dot(f, mar_ref[...], preferred_element_type=jnp.float32)


def _proj(x, w, mal, mar):
    c = x.shape[1]
    return pl.pallas_call(
        _proj_body,
        out_shape=(
            jax.ShapeDtypeStruct((NP, D), jnp.float32),
            jax.ShapeDtypeStruct((NP, H), jnp.float32),
            jax.ShapeDtypeStruct((NP, H), jnp.float32),
        ),
    )(x, w, mal, mar)


def _att_body(a_ref, elt_ref, er_ref, feat_ref, b_ref, out_ref):
    a = a_ref[...]
    for h in range(H):
        el_row = elt_ref[h : h + 1, :]
        er_col = er_ref[:, h : h + 1]
        z = el_row + er_col
        t1 = jnp.exp(el_row) * jnp.exp(er_col)
        t2 = jnp.exp(0.2 * el_row) * jnp.exp(0.2 * er_col)
        t = a * jnp.where(z > 0, t1, t2)
        den = jnp.sum(t, axis=1, keepdims=True)
        acc = jnp.dot(t, feat_ref[:, h * HF : (h + 1) * HF],
                      preferred_element_type=jnp.float32)
        o = acc / (den + 1e-9) + b_ref[:, h * HF : (h + 1) * HF]
        out_ref[:, h * HF : (h + 1) * HF] = jnp.maximum(o, 0.0)


def _att(a, elt, er, feat, b):
    return pl.pallas_call(
        _att_body,
        out_shape=jax.ShapeDtypeStruct((NP, D), jnp.float32),
    )(a, elt, er, feat, b)


def _head_matrix(al):
    eye = jnp.eye(H, dtype=al.dtype)
    return (al[:, :, None] * eye[:, None, :]).reshape(D, H)


def kernel(in_feat, edge_index, W_m, b_m, W_d, b_d, W1, al1, ar1, b1, W2, al2, ar2, b2):
    src = edge_index[0]
    dst = edge_index[1]
    adj = _build_adj_fn()(src, dst).reshape(NP, NP)

    bf = jnp.zeros((NP, 512), jnp.float32).at[:N, :495].set(in_feat)
    wm_p = jnp.zeros((512, FC), jnp.float32).at[:495].set(W_m)
    wd_p = jnp.zeros((512, FC), jnp.float32).at[:383].set(W_d)
    x = _fc(bf, wm_p, wd_p, b_m.reshape(1, FC), b_d.reshape(1, FC))

    feat1, el1, er1 = _proj(x, W1, _head_matrix(al1), _head_matrix(ar1))
    h1 = _att(adj, el1.T, er1, feat1, b1.reshape(1, D))

    feat2, el2, er2 = _proj(h1, W2, _head_matrix(al2), _head_matrix(ar2))
    h2 = _att(adj, el2.T, er2, feat2, b2.reshape(1, D))
    return h2[:N, :]


# speedup vs baseline: 71.0810x; 1.1072x over previous
import functools

import jax
import jax.numpy as jnp
from jax import lax
from jax.experimental import pallas as pl
from jax.experimental.pallas import tpu as pltpu
from jax.experimental.pallas import tpu_sc as plsc

N = 878
NP = 896
H = 8
HF = 256
FC = 256
E = 28096
D = H * HF

_ROWS_PER_TILE = NP // 32
_TILE_WORDS = _ROWS_PER_TILE * NP
_LANES = 16
_UNROLL = 4


def _adj_body(ei_hbm, zero_hbm, a_hbm, src_v, dst_v, acc_v):
    wid = lax.axis_index("s") * 2 + lax.axis_index("c")
    lo = wid * _ROWS_PER_TILE

    pltpu.sync_copy(ei_hbm.at[0], src_v)
    pltpu.sync_copy(ei_hbm.at[1], dst_v)
    pltpu.sync_copy(zero_hbm, acc_v)

    ones = jnp.ones((_LANES,), jnp.float32)
    nrows = jnp.uint32(_ROWS_PER_TILE)

    def _scatter(i, _):
        for k in range(_UNROLL):
            off = (i * _UNROLL + k) * _LANES
            d16 = dst_v[pl.ds(off, _LANES)]
            s16 = src_v[pl.ds(off, _LANES)]
            rel = d16 - lo
            msk = lax.convert_element_type(rel, jnp.uint32) < nrows
            plsc.addupdate_scatter(acc_v, [rel * NP + s16], ones, mask=msk)
        return 0

    lax.fori_loop(0, E // (_LANES * _UNROLL), _scatter, 0)

    pltpu.sync_copy(acc_v, a_hbm.at[pl.ds(wid * _TILE_WORDS, _TILE_WORDS)])


@functools.cache
def _build_adj_fn():
    return functools.partial(
        pl.kernel,
        out_type=jax.ShapeDtypeStruct((NP * NP,), jnp.float32),
        mesh=plsc.VectorSubcoreMesh(core_axis_name="c", subcore_axis_name="s"),
        compiler_params=pltpu.CompilerParams(needs_layout_passes=False),
        scratch_types=[
            pltpu.VMEM((E,), jnp.int32),
            pltpu.VMEM((E,), jnp.int32),
            pltpu.VMEM((_TILE_WORDS,), jnp.float32),
        ],
    )(_adj_body)


def _fc_body(bf_ref, wm_ref, wd_ref, bm_ref, bd_ref, x_ref):
    b = bf_ref[...]
    xm = jnp.dot(b, wm_ref[...], preferred_element_type=jnp.float32) + bm_ref[...]
    xd = jnp.dot(b, wd_ref[...], preferred_element_type=jnp.float32) + bd_ref[...]
    row = lax.broadcasted_iota(jnp.int32, (NP, 1), 0)
    x = jnp.where(row < 495, xm, xd)
    x_ref[...] = jnp.where(row < N, x, 0.0)


def _fc(bf, wm, wd, bm, bd):
    return pl.pallas_call(
        _fc_body,
        out_shape=jax.ShapeDtypeStruct((NP, FC), jnp.float32),
    )(bf, wm, wd, bm, bd)


def _proj_body(x_ref, w_ref, alf_ref, arf_ref, feat_ref, elt_ref, er_ref):
    f = jnp.dot(x_ref[...], w_ref[...], preferred_element_type=jnp.float32)
    feat_ref[...] = f
    y1 = f * alf_ref[...]
    y2 = f * arf_ref[...]
    el_cols = [jnp.sum(y1[:, h * HF : (h + 1) * HF], axis=1, keepdims=True)
               for h in range(H)]
    elt_ref[...] = jnp.concatenate(el_cols, axis=1).T
    for h in range(H):
        er_ref[:, h : h + 1] = jnp.sum(
            y2[:, h * HF : (h + 1) * HF], axis=1, keepdims=True)


def _proj(x, w, alf, arf):
    return pl.pallas_call(
        _proj_body,
        out_shape=(
            jax.ShapeDtypeStruct((NP, D), jnp.float32),
            jax.ShapeDtypeStruct((H, NP), jnp.float32),
            jax.ShapeDtypeStruct((NP, H), jnp.float32),
        ),
    )(x, w, alf, arf)


def _att_body(out_rows, a_ref, elt_ref, er_ref, feat_ref, b_ref, out_ref):
    a = a_ref[...]
    for h in range(H):
        el_row = elt_ref[h : h + 1, :]
        er_col = er_ref[:, h : h + 1]
        z = el_row + er_col
        t1 = jnp.exp(el_row) * jnp.exp(er_col)
        t2 = jnp.exp(0.2 * el_row) * jnp.exp(0.2 * er_col)
        t = a * jnp.where(z > 0, t1, t2)
        den = jnp.sum(t, axis=1, keepdims=True)
        acc = jnp.dot(t, feat_ref[:, h * HF : (h + 1) * HF],
                      preferred_element_type=jnp.float32)
        o = acc / (den + 1e-9) + b_ref[:, h * HF : (h + 1) * HF]
        out_ref[:, h * HF : (h + 1) * HF] = jnp.maximum(o, 0.0)[:out_rows, :]


def _att(a, elt, er, feat, b, out_rows=NP):
    return pl.pallas_call(
        functools.partial(_att_body, out_rows),
        out_shape=jax.ShapeDtypeStruct((out_rows, D), jnp.float32),
    )(a, elt, er, feat, b)


def kernel(in_feat, edge_index, W_m, b_m, W_d, b_d, W1, al1, ar1, b1, W2, al2, ar2, b2):
    zeros_tile = jnp.zeros((_TILE_WORDS,), jnp.float32)
    adj = _build_adj_fn()(edge_index, zeros_tile).reshape(NP, NP)

    bf = jnp.zeros((NP, 512), jnp.float32).at[:N, :495].set(in_feat)
    wm_p = jnp.zeros((512, FC), jnp.float32).at[:495].set(W_m)
    wd_p = jnp.zeros((512, FC), jnp.float32).at[:383].set(W_d)
    x = _fc(bf, wm_p, wd_p, b_m.reshape(1, FC), b_d.reshape(1, FC))

    feat1, elt1, er1 = _proj(x, W1, al1.reshape(1, D), ar1.reshape(1, D))
    h1 = _att(adj, elt1, er1, feat1, b1.reshape(1, D))

    feat2, elt2, er2 = _proj(h1, W2, al2.reshape(1, D), ar2.reshape(1, D))
    return _att(adj, elt2, er2, feat2, b2.reshape(1, D), out_rows=N)


# speedup vs baseline: 75.6768x; 1.0647x over previous
import functools

import jax
import jax.numpy as jnp
from jax import lax
from jax.experimental import pallas as pl
from jax.experimental.pallas import tpu as pltpu
from jax.experimental.pallas import tpu_sc as plsc

N = 878
NP = 896
H = 8
HF = 256
FC = 256
E = 28096
D = H * HF

_ROWS_PER_TILE = NP // 32
_TILE_WORDS = _ROWS_PER_TILE * NP
_LANES = 16
_UNROLL = 4


def _adj_body(ei_hbm, zero_hbm, a_hbm, src_v, dst_v, acc_v, sem0, sem1, sem2):
    wid = lax.axis_index("s") * 2 + lax.axis_index("c")
    lo = wid * _ROWS_PER_TILE

    c0 = pltpu.async_copy(ei_hbm.at[0], src_v, sem0)
    c1 = pltpu.async_copy(ei_hbm.at[1], dst_v, sem1)
    c2 = pltpu.async_copy(zero_hbm, acc_v, sem2)
    c0.wait()
    c1.wait()
    c2.wait()

    ones = jnp.ones((_LANES,), jnp.float32)
    nrows = jnp.uint32(_ROWS_PER_TILE)

    @plsc.parallel_loop(0, E // _LANES, unroll=8)
    def _scatter(i):
        off = i * _LANES
        d16 = dst_v[pl.ds(off, _LANES)]
        s16 = src_v[pl.ds(off, _LANES)]
        rel = d16 - lo
        msk = lax.convert_element_type(rel, jnp.uint32) < nrows
        plsc.addupdate_scatter(acc_v, [rel * NP + s16], ones, mask=msk)

    pltpu.sync_copy(acc_v, a_hbm.at[pl.ds(wid * _TILE_WORDS, _TILE_WORDS)])


@functools.cache
def _build_adj_fn():
    return functools.partial(
        pl.kernel,
        out_type=jax.ShapeDtypeStruct((NP * NP,), jnp.float32),
        mesh=plsc.VectorSubcoreMesh(core_axis_name="c", subcore_axis_name="s"),
        compiler_params=pltpu.CompilerParams(needs_layout_passes=False),
        scratch_types=[
            pltpu.VMEM((E,), jnp.int32),
            pltpu.VMEM((E,), jnp.int32),
            pltpu.VMEM((_TILE_WORDS,), jnp.float32),
            pltpu.SemaphoreType.DMA,
            pltpu.SemaphoreType.DMA,
            pltpu.SemaphoreType.DMA,
        ],
    )(_adj_body)


def _fc_body(bf_ref, wm_ref, wd_ref, bm_ref, bd_ref, x_ref):
    b = bf_ref[...]
    xm = jnp.dot(b, wm_ref[...], preferred_element_type=jnp.float32) + bm_ref[...]
    xd = jnp.dot(b, wd_ref[...], preferred_element_type=jnp.float32) + bd_ref[...]
    row = lax.broadcasted_iota(jnp.int32, (NP, 1), 0)
    x = jnp.where(row < 495, xm, xd)
    x_ref[...] = jnp.where(row < N, x, 0.0)


def _fc(bf, wm, wd, bm, bd):
    return pl.pallas_call(
        _fc_body,
        out_shape=jax.ShapeDtypeStruct((NP, FC), jnp.float32),
    )(bf, wm, wd, bm, bd)


def _proj_body(x_ref, w_ref, alf_ref, arf_ref, feat_ref, elt_ref, er_ref):
    f = jnp.dot(x_ref[...].astype(jnp.bfloat16), w_ref[...],
                preferred_element_type=jnp.float32)
    feat_ref[...] = f
    y1 = f * alf_ref[...]
    y2 = f * arf_ref[...]
    el_cols = [jnp.sum(y1[:, h * HF : (h + 1) * HF], axis=1, keepdims=True)
               for h in range(H)]
    elt_ref[...] = jnp.concatenate(el_cols, axis=1).T
    for h in range(H):
        er_ref[:, h : h + 1] = jnp.sum(
            y2[:, h * HF : (h + 1) * HF], axis=1, keepdims=True)


def _proj(x, w, alf, arf):
    return pl.pallas_call(
        _proj_body,
        out_shape=(
            jax.ShapeDtypeStruct((NP, D), jnp.float32),
            jax.ShapeDtypeStruct((H, NP), jnp.float32),
            jax.ShapeDtypeStruct((NP, H), jnp.float32),
        ),
    )(x, w, alf, arf)


def _att_body(out_rows, out_dtype, a_ref, elt_ref, er_ref, feat_ref, b_ref, out_ref):
    a = a_ref[...]
    fb = feat_ref[...].astype(jnp.bfloat16)
    for h in range(H):
        el_row = elt_ref[h : h + 1, :]
        er_col = er_ref[:, h : h + 1]
        z = el_row + er_col
        t1 = jnp.exp(el_row) * jnp.exp(er_col)
        t2 = jnp.exp(0.2 * el_row) * jnp.exp(0.2 * er_col)
        t = a * jnp.where(z > 0, t1, t2)
        den = jnp.sum(t, axis=1, keepdims=True)
        acc = jnp.dot(t.astype(jnp.bfloat16), fb[:, h * HF : (h + 1) * HF],
                      preferred_element_type=jnp.float32)
        o = acc / (den + 1e-9) + b_ref[:, h * HF : (h + 1) * HF]
        out_ref[:, h * HF : (h + 1) * HF] = jnp.maximum(o, 0.0)[:out_rows, :].astype(out_dtype)


def _att(a, elt, er, feat, b, out_rows=NP, out_dtype=jnp.float32):
    return pl.pallas_call(
        functools.partial(_att_body, out_rows, out_dtype),
        out_shape=jax.ShapeDtypeStruct((out_rows, D), out_dtype),
    )(a, elt, er, feat, b)


def kernel(in_feat, edge_index, W_m, b_m, W_d, b_d, W1, al1, ar1, b1, W2, al2, ar2, b2):
    zeros_tile = jnp.zeros((_TILE_WORDS,), jnp.float32)
    adj = _build_adj_fn()(edge_index, zeros_tile).reshape(NP, NP)

    bf = jnp.zeros((NP, 512), jnp.float32).at[:N, :495].set(in_feat)
    wm_p = jnp.zeros((512, FC), jnp.float32).at[:495].set(W_m)
    wd_p = jnp.zeros((512, FC), jnp.float32).at[:383].set(W_d)
    x = _fc(bf, wm_p, wd_p, b_m.reshape(1, FC), b_d.reshape(1, FC))

    feat1, elt1, er1 = _proj(x, W1.astype(jnp.bfloat16),
                             al1.reshape(1, D), ar1.reshape(1, D))
    h1 = _att(adj, elt1, er1, feat1, b1.reshape(1, D), out_dtype=jnp.bfloat16)

    feat2, elt2, er2 = _proj(h1, W2.astype(jnp.bfloat16),
                             al2.reshape(1, D), ar2.reshape(1, D))
    return _att(adj, elt2, er2, feat2, b2.reshape(1, D), out_rows=N)
